# baseline (speedup 1.0000x reference)
import functools

import jax
import jax.numpy as jnp
from jax import lax
from jax.experimental import pallas as pl
from jax.experimental.pallas import tpu as pltpu
from jax.experimental.pallas import tpu_sc as plsc

HIDDEN = 128
NREG = HIDDEN // 16
LN_EPS = 1e-12
NC, NS = 2, 16
NW = NC * NS
CHUNK = 128
NBUF = 2


def _tree_sum(xs):
    while len(xs) > 1:
        xs = [a + b for a, b in zip(xs[::2], xs[1::2])] + (
            [xs[-1]] if len(xs) % 2 else [])
    return xs[0]


def _sc_body(n_tokens, seq_hbm, reg_hbm, res_hbm, type_hbm, gamma_hbm,
             beta_hbm, out_hbm, sidx_v, ridx_v, rows_v, out_v, type_v,
             gamma_v, beta_v, sem_in, sem_out):
    per_w = n_tokens // NW
    nch = per_w // CHUNK
    wid = lax.axis_index("s") * NC + lax.axis_index("c")
    base_w = wid * per_w

    pltpu.sync_copy(seq_hbm.at[pl.ds(base_w, per_w)], sidx_v)
    pltpu.sync_copy(type_hbm, type_v)
    pltpu.sync_copy(gamma_hbm, gamma_v)
    pltpu.sync_copy(beta_hbm, beta_v)

    g = [gamma_v[pl.ds(j * 16, 16)] for j in range(NREG)]
    b = [beta_v[pl.ds(j * 16, 16)] for j in range(NREG)]
    cols = [lax.iota(jnp.int32, 16) + (j * 16) for j in range(NREG)]
    h_splat = jnp.full((16,), HIDDEN, jnp.int32)
    lane_ids = [jnp.full((16,), q, jnp.int32) for q in range(16)]
    idx15 = jnp.full((16,), 15, jnp.int32)
    inv_h = jnp.full((16,), 1.0 / HIDDEN, jnp.float32)
    eps_v = jnp.full((16,), LN_EPS, jnp.float32)
    magic_v = jnp.full((16,), 0x5F3759DF, jnp.int32)
    one_v = jnp.full((16,), 1, jnp.int32)
    c15_v = jnp.full((16,), 1.5, jnp.float32)
    half_v = jnp.full((16,), 0.5, jnp.float32)

    def start_in(ci, bf):
        base = base_w + ci * CHUNK
        pltpu.async_copy(reg_hbm.at[pl.ds(base, CHUNK)], ridx_v.at[bf],
                         sem_in.at[bf])
        pltpu.async_copy(res_hbm.at[sidx_v.at[pl.ds(ci * CHUNK, CHUNK)]],
                         rows_v.at[bf], sem_in.at[bf])

    def wait_in(bf):
        pltpu.make_async_copy(reg_hbm.at[pl.ds(0, CHUNK)], ridx_v.at[bf],
                              sem_in.at[bf]).wait()
        pltpu.make_async_copy(res_hbm.at[pl.ds(0, CHUNK)], rows_v.at[bf],
                              sem_in.at[bf]).wait()

    def start_out(ci, bf):
        base = base_w + ci * CHUNK
        pltpu.async_copy(out_v.at[bf], out_hbm.at[pl.ds(base, CHUNK)],
                         sem_out.at[bf])

    def wait_out(bf):
        pltpu.make_async_copy(out_v.at[bf], out_hbm.at[pl.ds(0, CHUNK)],
                              sem_out.at[bf]).wait()

    def compute(bf):
        @plsc.parallel_loop(0, CHUNK // 16, unroll=2)
        def grp_body(gi):
            rvec = ridx_v[bf, pl.ds(gi * 16, 16)] * h_splat
            for lane in range(16):
                t = gi * 16 + lane
                rsplat = jnp.take_along_axis(
                    rvec, lane_ids[lane], axis=0,
                    mode="promise_in_bounds")
                v = []
                for j in range(NREG):
                    tv = plsc.load_gather(type_v, [rsplat + cols[j]])
                    rv = rows_v[bf, t, pl.ds(j * 16, 16)]
                    v.append(rv + tv)
                s = _tree_sum(v)
                sq = _tree_sum([x * x for x in v])
                stot = jnp.take_along_axis(
                    plsc.cumsum(s), idx15, axis=0,
                    mode="promise_in_bounds")
                qtot = jnp.take_along_axis(
                    plsc.cumsum(sq), idx15, axis=0,
                    mode="promise_in_bounds")
                mean_v = stot * inv_h
                var_v = qtot * inv_h - mean_v * mean_v
                x = var_v + eps_v
                xi = plsc.bitcast(x, jnp.int32)
                yi = magic_v - lax.shift_right_arithmetic(xi, one_v)
                y = plsc.bitcast(yi, jnp.float32)
                y = y * (c15_v - (half_v * x) * (y * y))
                for j in range(NREG):
                    out_v[bf, t, pl.ds(j * 16, 16)] = (
                        (v[j] - mean_v) * y) * g[j] + b[j]

    for bf in range(NBUF):
        start_in(bf, bf)

    def outer(ci2, carry):
        for bf in range(NBUF):
            ci = ci2 * NBUF + bf
            wait_in(bf)

            @pl.when(ci2 > 0)
            def _():
                wait_out(bf)

            compute(bf)
            start_out(ci, bf)

            @pl.when(ci < nch - NBUF)
            def _():
                start_in(ci + NBUF, bf)
        return carry

    lax.fori_loop(0, nch // NBUF, outer, 0)
    for bf in range(NBUF):
        wait_out(bf)


def kernel(seq_tokens, region_indices, residue_table, type_table, gamma,
           beta):
    bsz, seq_len = seq_tokens.shape
    n = bsz * seq_len
    per_w = n // NW
    seq_flat = seq_tokens.reshape(n).astype(jnp.int32)
    reg_flat = region_indices.reshape(n).astype(jnp.int32)
    n_type = type_table.shape[0]
    type_flat = type_table.reshape(n_type * HIDDEN)

    mesh = plsc.VectorSubcoreMesh(core_axis_name="c", subcore_axis_name="s")
    run = pl.kernel(
        functools.partial(_sc_body, n),
        out_type=jax.ShapeDtypeStruct((n, HIDDEN), jnp.float32),
        mesh=mesh,
        compiler_params=pltpu.CompilerParams(needs_layout_passes=False),
        scratch_types=[
            pltpu.VMEM((per_w,), jnp.int32),
            pltpu.VMEM((NBUF, CHUNK), jnp.int32),
            pltpu.VMEM((NBUF, CHUNK, HIDDEN), jnp.float32),
            pltpu.VMEM((NBUF, CHUNK, HIDDEN), jnp.float32),
            pltpu.VMEM((n_type * HIDDEN,), jnp.float32),
            pltpu.VMEM((HIDDEN,), jnp.float32),
            pltpu.VMEM((HIDDEN,), jnp.float32),
            pltpu.SemaphoreType.DMA((NBUF,)),
            pltpu.SemaphoreType.DMA((NBUF,)),
        ],
    )
    out = run(seq_flat, reg_flat, residue_table, type_flat, gamma, beta)
    return out.reshape(bsz, seq_len, HIDDEN)


# speedup vs baseline: 2.4482x; 2.4482x over previous
import functools

import jax
import jax.numpy as jnp
from jax import lax
from jax.experimental import pallas as pl
from jax.experimental.pallas import tpu as pltpu
from jax.experimental.pallas import tpu_sc as plsc

HIDDEN = 128
NREG = HIDDEN // 16
LN_EPS = 1e-12
NC, NS = 2, 16
NW = NC * NS
CHUNK = 128
NBUF = 2


def _tree_sum(xs):
    while len(xs) > 1:
        xs = [a + b for a, b in zip(xs[::2], xs[1::2])] + (
            [xs[-1]] if len(xs) % 2 else [])
    return xs[0]


def _sc_body(n_tokens, seq_hbm, reg_hbm, res_hbm, type_hbm, gamma_hbm,
             beta_hbm, out_hbm, sidx_v, ridx_v, rows_v, out_v, type_v,
             gamma_v, beta_v, sem_in, sem_out):
    per_w = n_tokens // NW
    nch = per_w // CHUNK
    wid = lax.axis_index("s") * NC + lax.axis_index("c")
    base_w = wid * per_w

    pltpu.sync_copy(seq_hbm.at[pl.ds(base_w, per_w)], sidx_v)
    pltpu.sync_copy(type_hbm, type_v)
    pltpu.sync_copy(gamma_hbm, gamma_v)
    pltpu.sync_copy(beta_hbm, beta_v)

    g = [gamma_v[pl.ds(j * 16, 16)] for j in range(NREG)]
    b = [beta_v[pl.ds(j * 16, 16)] for j in range(NREG)]
    cols = [lax.iota(jnp.int32, 16) + (j * 16) for j in range(NREG)]
    h_splat = jnp.full((16,), HIDDEN, jnp.int32)
    lane_ids = [jnp.full((16,), q, jnp.int32) for q in range(16)]
    idx15 = jnp.full((16,), 15, jnp.int32)
    inv_h = jnp.full((16,), 1.0 / HIDDEN, jnp.float32)
    eps_v = jnp.full((16,), LN_EPS, jnp.float32)
    magic_v = jnp.full((16,), 0x5F3759DF, jnp.int32)
    one_v = jnp.full((16,), 1, jnp.int32)
    c15_v = jnp.full((16,), 1.5, jnp.float32)
    half_v = jnp.full((16,), 0.5, jnp.float32)

    def start_in(ci, bf):
        base = base_w + ci * CHUNK
        pltpu.async_copy(reg_hbm.at[pl.ds(base, CHUNK)], ridx_v.at[bf],
                         sem_in.at[bf])
        pltpu.async_copy(res_hbm.at[sidx_v.at[pl.ds(ci * CHUNK, CHUNK)]],
                         rows_v.at[bf], sem_in.at[bf])

    def wait_in(bf):
        pltpu.make_async_copy(reg_hbm.at[pl.ds(0, CHUNK)], ridx_v.at[bf],
                              sem_in.at[bf]).wait()
        pltpu.make_async_copy(res_hbm.at[pl.ds(0, CHUNK)], rows_v.at[bf],
                              sem_in.at[bf]).wait()

    def start_out(ci, bf):
        base = base_w + ci * CHUNK
        pltpu.async_copy(out_v.at[bf], out_hbm.at[pl.ds(base, CHUNK)],
                         sem_out.at[bf])

    def wait_out(bf):
        pltpu.make_async_copy(out_v.at[bf], out_hbm.at[pl.ds(0, CHUNK)],
                              sem_out.at[bf]).wait()

    def compute(bf):
        @plsc.parallel_loop(0, CHUNK // 16, unroll=1)
        def grp_body(gi):
            rvec = ridx_v[bf, pl.ds(gi * 16, 16)] * h_splat
            for lane in range(16):
                t = gi * 16 + lane
                rsplat = jnp.take_along_axis(
                    rvec, lane_ids[lane], axis=0,
                    mode="promise_in_bounds")
                v = []
                for j in range(NREG):
                    tv = plsc.load_gather(type_v, [rsplat + cols[j]])
                    rv = rows_v[bf, t, pl.ds(j * 16, 16)]
                    v.append(rv + tv)
                s = _tree_sum(v)
                sq = _tree_sum([x * x for x in v])
                stot = jnp.take_along_axis(
                    plsc.cumsum(s), idx15, axis=0,
                    mode="promise_in_bounds")
                qtot = jnp.take_along_axis(
                    plsc.cumsum(sq), idx15, axis=0,
                    mode="promise_in_bounds")
                mean_v = stot * inv_h
                var_v = qtot * inv_h - mean_v * mean_v
                x = var_v + eps_v
                xi = plsc.bitcast(x, jnp.int32)
                yi = magic_v - lax.shift_right_arithmetic(xi, one_v)
                y = plsc.bitcast(yi, jnp.float32)
                y = y * (c15_v - (half_v * x) * (y * y))
                for j in range(NREG):
                    out_v[bf, t, pl.ds(j * 16, 16)] = (
                        (v[j] - mean_v) * y) * g[j] + b[j]

    for bf in range(NBUF):
        start_in(bf, bf)

    def outer(ci2, carry):
        for bf in range(NBUF):
            ci = ci2 * NBUF + bf
            wait_in(bf)

            @pl.when(ci2 > 0)
            def _():
                wait_out(bf)

            compute(bf)
            start_out(ci, bf)

            @pl.when(ci < nch - NBUF)
            def _():
                start_in(ci + NBUF, bf)
        return carry

    lax.fori_loop(0, nch // NBUF, outer, 0)
    for bf in range(NBUF):
        wait_out(bf)


def kernel(seq_tokens, region_indices, residue_table, type_table, gamma,
           beta):
    bsz, seq_len = seq_tokens.shape
    n = bsz * seq_len
    per_w = n // NW
    seq_flat = seq_tokens.reshape(n).astype(jnp.int32)
    reg_flat = region_indices.reshape(n).astype(jnp.int32)
    n_type = type_table.shape[0]
    type_flat = type_table.reshape(n_type * HIDDEN)

    mesh = plsc.VectorSubcoreMesh(core_axis_name="c", subcore_axis_name="s")
    run = pl.kernel(
        functools.partial(_sc_body, n),
        out_type=jax.ShapeDtypeStruct((n, HIDDEN), jnp.float32),
        mesh=mesh,
        compiler_params=pltpu.CompilerParams(needs_layout_passes=False),
        scratch_types=[
            pltpu.VMEM((per_w,), jnp.int32),
            pltpu.VMEM((NBUF, CHUNK), jnp.int32),
            pltpu.VMEM((NBUF, CHUNK, HIDDEN), jnp.float32),
            pltpu.VMEM((NBUF, CHUNK, HIDDEN), jnp.float32),
            pltpu.VMEM((n_type * HIDDEN,), jnp.float32),
            pltpu.VMEM((HIDDEN,), jnp.float32),
            pltpu.VMEM((HIDDEN,), jnp.float32),
            pltpu.SemaphoreType.DMA((NBUF,)),
            pltpu.SemaphoreType.DMA((NBUF,)),
        ],
    )
    out = run(seq_flat, reg_flat, residue_table, type_flat, gamma, beta)
    return out.reshape(bsz, seq_len, HIDDEN)
